# baseline (speedup 1.0000x reference)
import functools

import jax
import jax.numpy as jnp
from jax import lax
from jax.experimental import pallas as pl
from jax.experimental.pallas import tpu as pltpu
from jax.experimental.pallas import tpu_sc as plsc

N = 10000
E = 320000
EMB = 128
HID = 100
D = 128

NC = 2
NS = 16
NW = NC * NS

GC = 80
NGC = N // GC

EC = 96
CPT = 108
EPT = CPT * EC
E_PAD = NW * EPT

ZR = 80
NZC = N // ZR

_mesh = plsc.VectorSubcoreMesh(core_axis_name="c", subcore_axis_name="s")


def _mm_body(x_ref, w_ref, o_ref):
    o_ref[...] = jnp.dot(x_ref[...], w_ref[...],
                         preferred_element_type=jnp.float32)


def _matmul(emb_table, w1p):
    return pl.pallas_call(
        _mm_body,
        grid=(10,),
        in_specs=[
            pl.BlockSpec((N // 10, EMB), lambda i: (i, 0)),
            pl.BlockSpec((EMB, D), lambda i: (0, 0)),
        ],
        out_specs=pl.BlockSpec((N // 10, D), lambda i: (i, 0)),
        out_shape=jax.ShapeDtypeStruct((N, D), jnp.float32),
    )(emb_table, w1p)


@functools.partial(
    pl.kernel,
    out_type=jax.ShapeDtypeStruct((N, D), jnp.float32),
    mesh=_mesh,
    scratch_types=[
        pltpu.VMEM((GC,), jnp.int32),
        pltpu.VMEM((GC, D), jnp.float32),
        pltpu.SemaphoreType.DMA,
    ],
)
def _sc_gather(tp_hbm, idx_hbm, out_hbm, idx_v, rows_v, sem):
    wid = lax.axis_index("s") * NC + lax.axis_index("c")

    def chunk(k, _):
        c = wid + k * NW

        @pl.when(c < NGC)
        def _():
            base = c * GC
            pltpu.sync_copy(idx_hbm.at[pl.ds(base, GC)], idx_v)
            pltpu.async_copy(tp_hbm.at[idx_v], rows_v, sem).wait()
            pltpu.sync_copy(rows_v, out_hbm.at[pl.ds(base, GC)])

        return 0

    lax.fori_loop(0, (NGC + NW - 1) // NW, chunk, 0)


NRB = 3
NIB = 4


@functools.partial(
    pl.kernel,
    out_type=jax.ShapeDtypeStruct((NC, N, D), jnp.float32),
    mesh=_mesh,
    scratch_types=[
        pltpu.VMEM_SHARED((N, D), jnp.float32),
        [pltpu.VMEM((EC,), jnp.int32) for _ in range(NIB)],
        [pltpu.VMEM((EC,), jnp.int32) for _ in range(NIB)],
        [pltpu.VMEM((EC,), jnp.float32) for _ in range(NIB)],
        [pltpu.VMEM((EC, D), jnp.float32) for _ in range(NRB)],
        [pltpu.SemaphoreType.DMA for _ in range(NRB)],
        [pltpu.SemaphoreType.DMA for _ in range(NRB)],
        [pltpu.SemaphoreType.DMA for _ in range(NIB)],
    ],
)
def _sc_edges(sup_hbm, src_hbm, dst_hbm, val_hbm, out_hbm,
              accum, srcb, dstb, valb, rows, gsem, ssem, isem):
    cid = lax.axis_index("c")
    sid = lax.axis_index("s")
    wid = sid * NC + cid
    ebase = wid * EPT

    def start_idx(k, i):
        base = ebase + k * EC
        pltpu.async_copy(src_hbm.at[pl.ds(base, EC)], srcb[i], isem[i])
        pltpu.async_copy(dst_hbm.at[pl.ds(base, EC)], dstb[i], isem[i])
        pltpu.async_copy(val_hbm.at[pl.ds(base, EC)], valb[i], isem[i])

    def wait_idx(i):
        pltpu.make_async_copy(src_hbm.at[pl.ds(0, EC)], srcb[i], isem[i]).wait()
        pltpu.make_async_copy(dst_hbm.at[pl.ds(0, EC)], dstb[i], isem[i]).wait()
        pltpu.make_async_copy(val_hbm.at[pl.ds(0, EC)], valb[i], isem[i]).wait()

    def start_gather(i, b):
        pltpu.async_copy(sup_hbm.at[srcb[i]], rows[b], gsem[b])

    def wait_gather(b):
        pltpu.make_async_copy(sup_hbm.at[srcb[0]], rows[b], gsem[b]).wait()

    def wait_scatter(b):
        pltpu.make_async_copy(rows[b], accum.at[dstb[0]], ssem[b]).wait()

    start_idx(0, 0)
    start_idx(1, 1)

    zv = jnp.zeros((16,), jnp.float32)

    def zrow(r, _):
        for j in range(D // 16):
            rows[0][r, pl.ds(16 * j, 16)] = zv
        return 0

    lax.fori_loop(0, ZR, zrow, 0)

    def zcopy(k, _):
        c = sid + k * NS

        @pl.when(c < NZC)
        def _():
            pltpu.sync_copy(rows[0].at[pl.ds(0, ZR)], accum.at[pl.ds(c * ZR, ZR)])

        return 0

    lax.fori_loop(0, (NZC + NS - 1) // NS, zcopy, 0)
    plsc.subcore_barrier()

    wait_idx(0)
    start_gather(0, 0)

    def step(p, _):
        for u in range(12):
            k = 12 * p + u
            b = u % NRB
            i = u % NIB
            b1 = (u + 1) % NRB
            i1 = (u + 1) % NIB
            i2 = (u + 2) % NIB

            wait_gather(b)

            @pl.when(k + 1 < CPT)
            def _():
                wait_idx(i1)

                @pl.when(k >= 2)
                def _():
                    wait_scatter(b1)

                start_gather(i1, b1)

            @pl.when(k + 2 < CPT)
            def _():
                start_idx(k + 2, i2)

            def scale(g, _):
                vv = valb[i][pl.ds(16 * g, 16)]
                for t in range(16):
                    v = vv[t]
                    r = 16 * g + t
                    for j in range(D // 16):
                        sl = pl.ds(16 * j, 16)
                        rows[b][r, sl] = rows[b][r, sl] * v
                return 0

            lax.fori_loop(0, EC // 16, scale, 0)
            pltpu.async_copy(rows[b], accum.at[dstb[i]], ssem[b], add=True)

        return 0

    lax.fori_loop(0, CPT // 12, step, 0)

    for b in range(NRB):
        wait_scatter(b)
    plsc.subcore_barrier()

    def wcopy(k, _):
        c = sid + k * NS

        @pl.when(c < NZC)
        def _():
            base = c * ZR
            pltpu.sync_copy(accum.at[pl.ds(base, ZR)],
                            out_hbm.at[cid, pl.ds(base, ZR)])

        return 0

    lax.fori_loop(0, (NZC + NS - 1) // NS, wcopy, 0)


def _comb_body(a_ref, b_ref, bias_ref, o_ref):
    o_ref[...] = a_ref[...] + b_ref[...] + bias_ref[...]


def _combine(p0, p1, b1p):
    return pl.pallas_call(
        _comb_body,
        grid=(10,),
        in_specs=[
            pl.BlockSpec((N // 10, D), lambda i: (i, 0)),
            pl.BlockSpec((N // 10, D), lambda i: (i, 0)),
            pl.BlockSpec((1, D), lambda i: (0, 0)),
        ],
        out_specs=pl.BlockSpec((N // 10, D), lambda i: (i, 0)),
        out_shape=jax.ShapeDtypeStruct((N, D), jnp.float32),
    )(p0, p1, b1p)


def kernel(hyperneigh, adj_src, adj_dst, adj_vals, emb_table, W1, b1):
    w1p = jnp.pad(W1, ((0, 0), (0, D - HID)))
    b1p = jnp.pad(b1, (0, D - HID)).reshape(1, D)

    pad = E_PAD - E
    fill = jnp.arange(pad, dtype=jnp.int32) % N
    src_p = jnp.concatenate([adj_src.astype(jnp.int32), fill])
    dst_p = jnp.concatenate([adj_dst.astype(jnp.int32), fill])
    val_p = jnp.pad(adj_vals, (0, pad))

    tp = _matmul(emb_table, w1p)
    support = _sc_gather(tp, hyperneigh.astype(jnp.int32))
    partials = _sc_edges(support, src_p, dst_p, val_p)
    out = _combine(partials[0], partials[1], b1p)
    return out[:, :HID]


# speedup vs baseline: 1.0571x; 1.0571x over previous
import functools

import jax
import jax.numpy as jnp
from jax import lax
from jax.experimental import pallas as pl
from jax.experimental.pallas import tpu as pltpu
from jax.experimental.pallas import tpu_sc as plsc

N = 10000
E = 320000
EMB = 128
HID = 100
D = 128

NC = 2
NS = 16
NW = NC * NS

GC = 80
NGC = N // GC

EC = 112
CPT = 90
EPT = CPT * EC
E_PAD = NW * EPT

ZR = 80
NZC = N // ZR

_mesh = plsc.VectorSubcoreMesh(core_axis_name="c", subcore_axis_name="s")


def _mm_body(x_ref, w_ref, o_ref):
    o_ref[...] = jnp.dot(x_ref[...], w_ref[...],
                         preferred_element_type=jnp.float32)


def _matmul(emb_table, w1p):
    return pl.pallas_call(
        _mm_body,
        grid=(10,),
        in_specs=[
            pl.BlockSpec((N // 10, EMB), lambda i: (i, 0)),
            pl.BlockSpec((EMB, D), lambda i: (0, 0)),
        ],
        out_specs=pl.BlockSpec((N // 10, D), lambda i: (i, 0)),
        out_shape=jax.ShapeDtypeStruct((N, D), jnp.float32),
    )(emb_table, w1p)


@functools.partial(
    pl.kernel,
    out_type=jax.ShapeDtypeStruct((N, D), jnp.float32),
    mesh=_mesh,
    scratch_types=[
        pltpu.VMEM((GC,), jnp.int32),
        pltpu.VMEM((GC, D), jnp.float32),
        pltpu.SemaphoreType.DMA,
    ],
)
def _sc_gather(tp_hbm, idx_hbm, out_hbm, idx_v, rows_v, sem):
    wid = lax.axis_index("s") * NC + lax.axis_index("c")

    def chunk(k, _):
        c = wid + k * NW

        @pl.when(c < NGC)
        def _():
            base = c * GC
            pltpu.sync_copy(idx_hbm.at[pl.ds(base, GC)], idx_v)
            pltpu.async_copy(tp_hbm.at[idx_v], rows_v, sem).wait()
            pltpu.sync_copy(rows_v, out_hbm.at[pl.ds(base, GC)])

        return 0

    lax.fori_loop(0, (NGC + NW - 1) // NW, chunk, 0)


NRB = 3
NIB = 4


@functools.partial(
    pl.kernel,
    out_type=jax.ShapeDtypeStruct((NC, N, D), jnp.float32),
    mesh=_mesh,
    scratch_types=[
        pltpu.VMEM_SHARED((N, D), jnp.float32),
        [pltpu.VMEM((EC,), jnp.int32) for _ in range(NIB)],
        [pltpu.VMEM((EC,), jnp.int32) for _ in range(NIB)],
        [pltpu.VMEM((EC,), jnp.float32) for _ in range(NIB)],
        [pltpu.VMEM((EC, D), jnp.float32) for _ in range(NRB)],
        [pltpu.SemaphoreType.DMA for _ in range(NRB)],
        [pltpu.SemaphoreType.DMA for _ in range(NRB)],
        [pltpu.SemaphoreType.DMA for _ in range(NIB)],
    ],
)
def _sc_edges(sup_hbm, src_hbm, dst_hbm, val_hbm, out_hbm,
              accum, srcb, dstb, valb, rows, gsem, ssem, isem):
    cid = lax.axis_index("c")
    sid = lax.axis_index("s")
    wid = sid * NC + cid
    ebase = wid * EPT

    def start_idx(k, i):
        base = ebase + k * EC
        pltpu.async_copy(src_hbm.at[pl.ds(base, EC)], srcb[i], isem[i])
        pltpu.async_copy(dst_hbm.at[pl.ds(base, EC)], dstb[i], isem[i])
        pltpu.async_copy(val_hbm.at[pl.ds(base, EC)], valb[i], isem[i])

    def wait_idx(i):
        pltpu.make_async_copy(src_hbm.at[pl.ds(0, EC)], srcb[i], isem[i]).wait()
        pltpu.make_async_copy(dst_hbm.at[pl.ds(0, EC)], dstb[i], isem[i]).wait()
        pltpu.make_async_copy(val_hbm.at[pl.ds(0, EC)], valb[i], isem[i]).wait()

    def start_gather(i, b):
        pltpu.async_copy(sup_hbm.at[srcb[i]], rows[b], gsem[b])

    def wait_gather(b):
        pltpu.make_async_copy(sup_hbm.at[srcb[0]], rows[b], gsem[b]).wait()

    def wait_scatter(b):
        pltpu.make_async_copy(rows[b], accum.at[dstb[0]], ssem[b]).wait()

    start_idx(0, 0)
    start_idx(1, 1)

    zv = jnp.zeros((16,), jnp.float32)

    def zrow(r, _):
        for j in range(D // 16):
            rows[0][r, pl.ds(16 * j, 16)] = zv
        return 0

    lax.fori_loop(0, ZR, zrow, 0)

    def zcopy(k, _):
        c = sid + k * NS

        @pl.when(c < NZC)
        def _():
            pltpu.sync_copy(rows[0].at[pl.ds(0, ZR)], accum.at[pl.ds(c * ZR, ZR)])

        return 0

    lax.fori_loop(0, (NZC + NS - 1) // NS, zcopy, 0)
    plsc.subcore_barrier()

    wait_idx(0)
    start_gather(0, 0)

    def step(p, _):
        for u in range(12):
            k = 12 * p + u
            b = u % NRB
            i = u % NIB
            b1 = (u + 1) % NRB
            i1 = (u + 1) % NIB
            i2 = (u + 2) % NIB

            @pl.when(k < CPT)
            def _():
                wait_gather(b)

                @pl.when(k + 1 < CPT)
                def _():
                    wait_idx(i1)

                    @pl.when(k >= 2)
                    def _():
                        wait_scatter(b1)

                    start_gather(i1, b1)

                @pl.when(k + 2 < CPT)
                def _():
                    start_idx(k + 2, i2)

                def scale(g, _):
                    vv = valb[i][pl.ds(16 * g, 16)]
                    for t in range(16):
                        v = vv[t]
                        r = 16 * g + t
                        for j in range(D // 16):
                            sl = pl.ds(16 * j, 16)
                            rows[b][r, sl] = rows[b][r, sl] * v
                    return 0

                lax.fori_loop(0, EC // 16, scale, 0)
                pltpu.async_copy(rows[b], accum.at[dstb[i]], ssem[b], add=True)

        return 0

    lax.fori_loop(0, (CPT + 11) // 12, step, 0)

    for b in range(NRB):
        wait_scatter(b)
    plsc.subcore_barrier()

    def wcopy(k, _):
        c = sid + k * NS

        @pl.when(c < NZC)
        def _():
            base = c * ZR
            pltpu.sync_copy(accum.at[pl.ds(base, ZR)],
                            out_hbm.at[cid, pl.ds(base, ZR)])

        return 0

    lax.fori_loop(0, (NZC + NS - 1) // NS, wcopy, 0)


def _comb_body(a_ref, b_ref, bias_ref, o_ref):
    o_ref[...] = a_ref[...] + b_ref[...] + bias_ref[...]


def _combine(p0, p1, b1p):
    return pl.pallas_call(
        _comb_body,
        grid=(10,),
        in_specs=[
            pl.BlockSpec((N // 10, D), lambda i: (i, 0)),
            pl.BlockSpec((N // 10, D), lambda i: (i, 0)),
            pl.BlockSpec((1, D), lambda i: (0, 0)),
        ],
        out_specs=pl.BlockSpec((N // 10, D), lambda i: (i, 0)),
        out_shape=jax.ShapeDtypeStruct((N, D), jnp.float32),
    )(p0, p1, b1p)


def kernel(hyperneigh, adj_src, adj_dst, adj_vals, emb_table, W1, b1):
    w1p = jnp.pad(W1, ((0, 0), (0, D - HID)))
    b1p = jnp.pad(b1, (0, D - HID)).reshape(1, D)

    pad = E_PAD - E
    fill = jnp.arange(pad, dtype=jnp.int32) % N
    src_p = jnp.concatenate([adj_src.astype(jnp.int32), fill])
    dst_p = jnp.concatenate([adj_dst.astype(jnp.int32), fill])
    val_p = jnp.pad(adj_vals, (0, pad))

    tp = _matmul(emb_table, w1p)
    support = _sc_gather(tp, hyperneigh.astype(jnp.int32))
    partials = _sc_edges(support, src_p, dst_p, val_p)
    out = _combine(partials[0], partials[1], b1p)
    return out[:, :HID]


# speedup vs baseline: 1.0797x; 1.0214x over previous
import functools

import jax
import jax.numpy as jnp
from jax import lax
from jax.experimental import pallas as pl
from jax.experimental.pallas import tpu as pltpu
from jax.experimental.pallas import tpu_sc as plsc

N = 10000
E = 320000
EMB = 128
HID = 100
D = 128

NC = 2
NS = 16
NW = NC * NS

GC = 80
NGC = N // GC

EC = 128
CPT = 80
EPT = CPT * EC
E_PAD = NW * EPT

ZR = 80
NZC = N // ZR

_mesh = plsc.VectorSubcoreMesh(core_axis_name="c", subcore_axis_name="s")


def _mm_body(x_ref, w_ref, o_ref):
    o_ref[...] = jnp.dot(x_ref[...], w_ref[...],
                         preferred_element_type=jnp.float32)


def _matmul(emb_table, w1p):
    return pl.pallas_call(
        _mm_body,
        grid=(10,),
        in_specs=[
            pl.BlockSpec((N // 10, EMB), lambda i: (i, 0)),
            pl.BlockSpec((EMB, D), lambda i: (0, 0)),
        ],
        out_specs=pl.BlockSpec((N // 10, D), lambda i: (i, 0)),
        out_shape=jax.ShapeDtypeStruct((N, D), jnp.float32),
    )(emb_table, w1p)


NRB = 2
NIB = 3


@functools.partial(
    pl.kernel,
    out_type=jax.ShapeDtypeStruct((NC, N, D), jnp.float32),
    mesh=_mesh,
    scratch_types=[
        pltpu.VMEM_SHARED((N, D), jnp.float32),
        pltpu.VMEM_SHARED((N,), jnp.int32),
        [pltpu.VMEM((EC,), jnp.int32) for _ in range(NIB)],
        [pltpu.VMEM((EC,), jnp.int32) for _ in range(NIB)],
        [pltpu.VMEM((EC,), jnp.float32) for _ in range(NIB)],
        [pltpu.VMEM((EC,), jnp.int32) for _ in range(NRB)],
        [pltpu.VMEM((EC, D), jnp.float32) for _ in range(NRB)],
        [pltpu.SemaphoreType.DMA for _ in range(NRB)],
        [pltpu.SemaphoreType.DMA for _ in range(NRB)],
        [pltpu.SemaphoreType.DMA for _ in range(NIB)],
    ],
)
def _sc_edges(tp_hbm, hyp_hbm, src_hbm, dst_hbm, val_hbm, out_hbm,
              accum, hyp_v, srcb, dstb, valb, hsb, rows,
              gsem, ssem, isem):
    cid = lax.axis_index("c")
    sid = lax.axis_index("s")
    wid = sid * NC + cid
    ebase = wid * EPT

    def start_idx(k, i):
        base = ebase + k * EC
        pltpu.async_copy(src_hbm.at[pl.ds(base, EC)], srcb[i], isem[i])
        pltpu.async_copy(dst_hbm.at[pl.ds(base, EC)], dstb[i], isem[i])
        pltpu.async_copy(val_hbm.at[pl.ds(base, EC)], valb[i], isem[i])

    def wait_idx(i):
        pltpu.make_async_copy(src_hbm.at[pl.ds(0, EC)], srcb[i], isem[i]).wait()
        pltpu.make_async_copy(dst_hbm.at[pl.ds(0, EC)], dstb[i], isem[i]).wait()
        pltpu.make_async_copy(val_hbm.at[pl.ds(0, EC)], valb[i], isem[i]).wait()

    def start_gather(i, b):
        pltpu.sync_copy(hyp_v.at[srcb[i]], hsb[b])
        pltpu.async_copy(tp_hbm.at[hsb[b]], rows[b], gsem[b])

    def wait_gather(b):
        pltpu.make_async_copy(tp_hbm.at[hsb[0]], rows[b], gsem[b]).wait()

    def wait_scatter(b):
        pltpu.make_async_copy(rows[b], accum.at[dstb[0]], ssem[b]).wait()

    @pl.when(sid == 0)
    def _():
        pltpu.sync_copy(hyp_hbm, hyp_v)

    start_idx(0, 0)
    start_idx(1, 1)

    zv = jnp.zeros((16,), jnp.float32)

    def zrow(r, _):
        for j in range(D // 16):
            rows[0][r, pl.ds(16 * j, 16)] = zv
        return 0

    lax.fori_loop(0, ZR, zrow, 0)

    def zcopy(k, _):
        c = sid + k * NS

        @pl.when(c < NZC)
        def _():
            pltpu.sync_copy(rows[0].at[pl.ds(0, ZR)], accum.at[pl.ds(c * ZR, ZR)])

        return 0

    lax.fori_loop(0, (NZC + NS - 1) // NS, zcopy, 0)
    plsc.subcore_barrier()

    wait_idx(0)
    start_gather(0, 0)

    def step(p, _):
        for u in range(12):
            k = 12 * p + u
            b = u % NRB
            i = u % NIB
            b1 = (u + 1) % NRB
            i1 = (u + 1) % NIB
            i2 = (u + 2) % NIB

            @pl.when(k < CPT)
            def _():
                wait_gather(b)

                @pl.when(k + 1 < CPT)
                def _():
                    wait_idx(i1)

                    @pl.when(k + 1 >= NRB)
                    def _():
                        wait_scatter(b1)

                    start_gather(i1, b1)

                @pl.when(k + 2 < CPT)
                def _():
                    start_idx(k + 2, i2)

                def scale(g, _):
                    vv = valb[i][pl.ds(16 * g, 16)]
                    for t in range(16):
                        v = vv[t]
                        r = 16 * g + t
                        for j in range(D // 16):
                            sl = pl.ds(16 * j, 16)
                            rows[b][r, sl] = rows[b][r, sl] * v
                    return 0

                lax.fori_loop(0, EC // 16, scale, 0)
                pltpu.async_copy(rows[b], accum.at[dstb[i]], ssem[b], add=True)

        return 0

    lax.fori_loop(0, (CPT + 11) // 12, step, 0)

    for b in range(NRB):
        wait_scatter(b)
    plsc.subcore_barrier()

    def wcopy(k, _):
        c = sid + k * NS

        @pl.when(c < NZC)
        def _():
            base = c * ZR
            pltpu.sync_copy(accum.at[pl.ds(base, ZR)],
                            out_hbm.at[cid, pl.ds(base, ZR)])

        return 0

    lax.fori_loop(0, (NZC + NS - 1) // NS, wcopy, 0)


def _comb_body(a_ref, b_ref, bias_ref, o_ref):
    o_ref[...] = a_ref[...] + b_ref[...] + bias_ref[...]


def _combine(p0, p1, b1p):
    return pl.pallas_call(
        _comb_body,
        grid=(10,),
        in_specs=[
            pl.BlockSpec((N // 10, D), lambda i: (i, 0)),
            pl.BlockSpec((N // 10, D), lambda i: (i, 0)),
            pl.BlockSpec((1, D), lambda i: (0, 0)),
        ],
        out_specs=pl.BlockSpec((N // 10, D), lambda i: (i, 0)),
        out_shape=jax.ShapeDtypeStruct((N, D), jnp.float32),
    )(p0, p1, b1p)


def kernel(hyperneigh, adj_src, adj_dst, adj_vals, emb_table, W1, b1):
    w1p = jnp.pad(W1, ((0, 0), (0, D - HID)))
    b1p = jnp.pad(b1, (0, D - HID)).reshape(1, D)

    pad = E_PAD - E
    fill = jnp.arange(pad, dtype=jnp.int32) % N
    src_p = jnp.concatenate([adj_src.astype(jnp.int32), fill])
    dst_p = jnp.concatenate([adj_dst.astype(jnp.int32), fill])
    val_p = jnp.pad(adj_vals, (0, pad))

    tp = _matmul(emb_table, w1p)
    partials = _sc_edges(tp, hyperneigh.astype(jnp.int32), src_p, dst_p, val_p)
    out = _combine(partials[0], partials[1], b1p)
    return out[:, :HID]
